# baseline (speedup 1.0000x reference)
import jax
import jax.numpy as jnp
from jax import lax
from jax.experimental import pallas as pl
from jax.experimental.pallas import tpu as pltpu
from jax.experimental.pallas import tpu_sc as plsc

E = 256
N = 16384
NC = 2
NS = 16
NW = NC * NS
S = 4096
CPW = S // NW
CHUNK = 128
L = 16
G = CHUNK // L
TC_BLK = 1024
TC_GRID = (N - S) // TC_BLK


def _transform_slab(buf):
    zeros = tuple(jnp.zeros((L,), jnp.float32) for _ in range(G))

    @plsc.parallel_loop(1, E, carry=zeros, unroll=4)
    def accs(i, accs_in):
        return tuple(
            accs_in[g] + jnp.abs(buf[i, pl.ds(g * L, L)]) for g in range(G)
        )

    scales = []
    for g in range(G):
        sl = pl.ds(g * L, L)
        s1 = accs[g]
        x0 = buf[0, sl]
        lb = x0 - s1
        ub = x0 + s1
        crossing = (lb <= 0.0) & (ub >= 0.0)
        ub_le0 = ub <= 0.0
        alpha = 1.0 - lb
        scale = jnp.where(ub_le0, 0.0, jnp.where(crossing, alpha, 1.0))
        newc = alpha * x0 - alpha * lb * 0.5
        r0 = jnp.where(ub_le0, 0.0, jnp.where(crossing, newc, x0))
        buf[0, sl] = r0
        scales.append(scale)

    @plsc.parallel_loop(1, E, unroll=4)
    def _(i):
        for g in range(G):
            sl = pl.ds(g * L, L)
            buf[i, sl] = buf[i, sl] * scales[g]


def _tec_body(x_hbm, o_hbm, buf, sem_in, sem_out):
    wid = lax.axis_index("s") * NC + lax.axis_index("c")
    c0 = wid * CPW
    pltpu.async_copy(x_hbm.at[:, pl.ds(c0, CHUNK)], buf, sem_in).wait()
    _transform_slab(buf)
    pltpu.async_copy(buf, o_hbm.at[:, pl.ds(c0, CHUNK)], sem_out).wait()


def _sc_part(x):
    run = pl.kernel(
        _tec_body,
        out_type=jax.ShapeDtypeStruct((E, S), jnp.float32),
        mesh=plsc.VectorSubcoreMesh(core_axis_name="c", subcore_axis_name="s"),
        scratch_types=[
            pltpu.VMEM((E, CHUNK), jnp.float32),
            pltpu.SemaphoreType.DMA,
            pltpu.SemaphoreType.DMA,
        ],
    )
    return run(x)


def _tc_block(x_ref, o_ref):
    xb = x_ref[...]
    x0 = xb[0, :]
    s1 = jnp.sum(jnp.abs(xb), axis=0) - jnp.abs(x0)
    lb = x0 - s1
    ub = x0 + s1
    crossing = (lb <= 0.0) & (ub >= 0.0)
    ub_le0 = ub <= 0.0
    alpha = 1.0 - lb
    scale = jnp.where(ub_le0, 0.0, jnp.where(crossing, alpha, 1.0))
    newc = alpha * x0 - alpha * lb * 0.5
    r0 = jnp.where(ub_le0, 0.0, jnp.where(crossing, newc, x0))
    o_ref[...] = xb * scale[None, :]
    o_ref[0, :] = r0


def _tc_part(x):
    return pl.pallas_call(
        _tc_block,
        grid=(TC_GRID,),
        in_specs=[
            pl.BlockSpec((E, TC_BLK), lambda j: (0, j + S // TC_BLK)),
        ],
        out_specs=pl.BlockSpec((E, TC_BLK), lambda j: (0, j + S // TC_BLK)),
        out_shape=jax.ShapeDtypeStruct((E, N), jnp.float32),
        compiler_params=pltpu.CompilerParams(
            dimension_semantics=("parallel",),
        ),
    )(x)


def kernel(x):
    sc_out = _sc_part(x)
    tc_out = _tc_part(x)
    return lax.dynamic_update_slice(tc_out, sc_out, (0, 0))
